# baseline (speedup 1.0000x reference)
import functools

import jax
import jax.numpy as jnp
from jax import lax
from jax.experimental import pallas as pl
from jax.experimental.pallas import tpu as pltpu
from jax.experimental.pallas import tpu_sc as plsc

N_NODES = 100000
N_EDGES = 3200000
D_PAD = 32
DH = 16

NC = 2
NS = 16

ACC_ROWS = 102400
CH = 128
N_CH = 5
BLK = CH * N_CH
NBLK = N_EDGES // BLK
N_IDX_ROWS = N_EDGES // CH
ZCH = 128
OUT_CH = 1000
N_OUT_CHUNKS = N_NODES // OUT_CH


def _sc_aggregate(table3, edge3):
    mesh = plsc.VectorSubcoreMesh(core_axis_name="c", subcore_axis_name="s")

    @functools.partial(
        pl.kernel,
        out_type=jax.ShapeDtypeStruct((NC, N_NODES, DH), jnp.float32),
        mesh=mesh,
        scratch_types=[
            pltpu.VMEM_SHARED((ACC_ROWS, DH), jnp.float32),
            pltpu.VMEM((2, BLK), jnp.int32),
            pltpu.VMEM((2, BLK), jnp.int32),
            pltpu.VMEM((2, BLK, DH), jnp.float32),
            pltpu.SemaphoreType.DMA((2,)),
            pltpu.SemaphoreType.DMA((2,)),
            pltpu.SemaphoreType.DMA((2,)),
        ],
        compiler_params=pltpu.CompilerParams(use_tc_tiling_on_sc=False),
    )
    def agg(table, edges, out, acc, src_a, dst_a, rows_a, gsem, ssem, isem):
        c = lax.axis_index("c")
        s = lax.axis_index("s")
        tbl = table.at[c]

        zero = jnp.zeros((16,), jnp.float32)
        for r in range(ZCH):
            rows_a[0, r, pl.ds(0, 16)] = zero
        z0 = s * (ACC_ROWS // NS)

        def zloop(j, _):
            pltpu.sync_copy(
                rows_a.at[0, pl.ds(0, ZCH)], acc.at[pl.ds(z0 + j * ZCH, ZCH)]
            )
            return 0

        lax.fori_loop(0, (ACC_ROWS // NS) // ZCH, zloop, 0)
        plsc.subcore_barrier()

        nblocks = (NBLK - s + NS - 1) // NS

        pltpu.sync_copy(edges.at[0, pl.ds(s * BLK, BLK)], src_a.at[0])
        pltpu.sync_copy(edges.at[1, pl.ds(s * BLK, BLK)], dst_a.at[0])

        def outer(i, _):
            p = i & 1
            q = 1 - p
            gd = pltpu.async_copy(
                tbl.at[src_a.at[p]], rows_a.at[p], gsem.at[p]
            )

            @pl.when(i >= 1)
            def _():
                pltpu.make_async_copy(
                    rows_a.at[q], acc.at[dst_a.at[q]], ssem.at[q]
                ).wait()

            @pl.when(i + 1 < nblocks)
            def _():
                e0n = (s + (i + 1) * NS) * BLK
                pltpu.async_copy(
                    edges.at[0, pl.ds(e0n, BLK)], src_a.at[q], isem.at[q]
                )
                pltpu.async_copy(
                    edges.at[1, pl.ds(e0n, BLK)], dst_a.at[q], isem.at[q]
                )

            gd.wait()
            pltpu.async_copy(
                rows_a.at[p], acc.at[dst_a.at[p]], ssem.at[p], add=True
            )

            @pl.when(i + 1 < nblocks)
            def _():
                e0n = (s + (i + 1) * NS) * BLK
                pltpu.make_async_copy(
                    edges.at[0, pl.ds(e0n, BLK)], src_a.at[q], isem.at[q]
                ).wait()
                pltpu.make_async_copy(
                    edges.at[1, pl.ds(e0n, BLK)], dst_a.at[q], isem.at[q]
                ).wait()

            return 0

        lax.fori_loop(0, nblocks, outer, 0)
        last = (nblocks - 1) & 1
        pltpu.make_async_copy(
            rows_a.at[last], acc.at[dst_a.at[last]], ssem.at[last]
        ).wait()
        plsc.subcore_barrier()

        n_chunks = (N_OUT_CHUNKS - s + NS - 1) // NS

        def cloop(i, _):
            r0 = (s + i * NS) * OUT_CH
            pltpu.sync_copy(
                acc.at[pl.ds(r0, OUT_CH)], out.at[c, pl.ds(r0, OUT_CH)]
            )
            return 0

        lax.fori_loop(0, n_chunks, cloop, 0)

    return agg(table3, edge3)


_BN = 2000
_NBLK = N_NODES // _BN


def _mlp_body(h_ref, w1_ref, b1_ref, w2_ref, o_ref):
    h = jnp.concatenate([h_ref[0], h_ref[1]], axis=1)
    t = jnp.dot(h, w1_ref[...], preferred_element_type=jnp.float32)
    t = jnp.maximum(t + b1_ref[...], 0.0)
    y = jnp.dot(t, w2_ref[...], preferred_element_type=jnp.float32)
    o_ref[0] = y[:, :DH]
    o_ref[1] = y[:, DH:]


def _mlp(h1s, W1p, b1r, W2p):
    return pl.pallas_call(
        _mlp_body,
        grid=(_NBLK,),
        in_specs=[
            pl.BlockSpec((NC, _BN, DH), lambda i: (0, i, 0)),
            pl.BlockSpec((D_PAD, 48), lambda i: (0, 0)),
            pl.BlockSpec((1, 48), lambda i: (0, 0)),
            pl.BlockSpec((48, D_PAD), lambda i: (0, 0)),
        ],
        out_specs=pl.BlockSpec((NC, _BN, DH), lambda i: (0, i, 0)),
        out_shape=jax.ShapeDtypeStruct((NC, N_NODES, DH), jnp.float32),
    )(h1s, W1p, b1r, W2p)


def _stats_body(h_ref, b2_ref, sum_ref, sq_ref):
    i = pl.program_id(0)
    x = jnp.concatenate([h_ref[0], h_ref[1]], axis=1) + b2_ref[...]
    col = lax.broadcasted_iota(jnp.int32, (_BN, D_PAD), 1)
    xs = jnp.where(col < 24, x, 0.0)

    @pl.when(i == 0)
    def _():
        sum_ref[...] = jnp.zeros((1, 1), jnp.float32)
        sq_ref[...] = jnp.zeros((1, 1), jnp.float32)

    sum_ref[...] += jnp.full((1, 1), jnp.sum(xs), jnp.float32)
    sq_ref[...] += jnp.full((1, 1), jnp.sum(xs * xs), jnp.float32)


def _stats(h2s, b2r):
    return pl.pallas_call(
        _stats_body,
        grid=(_NBLK,),
        in_specs=[
            pl.BlockSpec((NC, _BN, DH), lambda i: (0, i, 0)),
            pl.BlockSpec((1, D_PAD), lambda i: (0, 0)),
        ],
        out_specs=[
            pl.BlockSpec((1, 1), lambda i: (0, 0)),
            pl.BlockSpec((1, 1), lambda i: (0, 0)),
        ],
        out_shape=[
            jax.ShapeDtypeStruct((1, 1), jnp.float32),
            jax.ShapeDtypeStruct((1, 1), jnp.float32),
        ],
    )(h2s, b2r)


def _apply_body(h_ref, b2_ref, mu_ref, inv_ref, o_ref):
    x = jnp.concatenate([h_ref[0], h_ref[1][:, :8]], axis=1)
    xb = x + b2_ref[...][:, :24]
    o_ref[...] = (xb - mu_ref[0, 0]) * inv_ref[0, 0]


def _apply(h2s, b2r, mu, inv):
    return pl.pallas_call(
        _apply_body,
        grid=(_NBLK,),
        in_specs=[
            pl.BlockSpec((NC, _BN, DH), lambda i: (0, i, 0)),
            pl.BlockSpec((1, D_PAD), lambda i: (0, 0)),
            pl.BlockSpec((1, 1), lambda i: (0, 0)),
            pl.BlockSpec((1, 1), lambda i: (0, 0)),
        ],
        out_specs=pl.BlockSpec((_BN, 24), lambda i: (i, 0)),
        out_shape=jax.ShapeDtypeStruct((N_NODES, 24), jnp.float32),
    )(h2s, b2r, mu, inv)


def kernel(features, edge_index, W1, b1, W2, b2):
    n, d = features.shape
    assert n == N_NODES and d == 24 and edge_index.shape == (2, N_EDGES)

    fpad = jnp.pad(features, ((0, 0), (0, D_PAD - 24)))
    ftab = jnp.stack([fpad[:, :DH], fpad[:, DH:]], axis=0)
    W1p = jnp.pad(W1, ((0, D_PAD - 24), (0, 0)))
    W2p = jnp.pad(W2, ((0, 0), (0, D_PAD - 24)))
    b1r = b1.reshape(1, 48)
    b2r = jnp.pad(b2, (0, D_PAD - 24)).reshape(1, D_PAD)

    edge3 = edge_index

    h1s = _sc_aggregate(ftab, edge3)
    ys = _mlp(h1s, W1p, b1r, W2p)
    h2s = _sc_aggregate(ys, edge3)

    ssum, ssq = _stats(h2s, b2r)
    cnt = jnp.float32(N_NODES * 24)
    mu = ssum / cnt
    var = (ssq - cnt * mu * mu) / (cnt - 1.0)
    inv = lax.rsqrt(var)
    return _apply(h2s, b2r, mu, inv)


# speedup vs baseline: 1.0893x; 1.0893x over previous
import functools

import jax
import jax.numpy as jnp
from jax import lax
from jax.experimental import pallas as pl
from jax.experimental.pallas import tpu as pltpu
from jax.experimental.pallas import tpu_sc as plsc

N_NODES = 100000
N_EDGES = 3200000
D_PAD = 32
DH = 16

NC = 2
NS = 16

ACC_ROWS = 102400
CH = 128
N_CH = 5
BLK = CH * N_CH
NBLK = N_EDGES // BLK
N_IDX_ROWS = N_EDGES // CH
ZCH = 128
OUT_CH = 1000
N_OUT_CHUNKS = N_NODES // OUT_CH


def _sc_aggregate(table3, edge3):
    mesh = plsc.VectorSubcoreMesh(core_axis_name="c", subcore_axis_name="s")

    @functools.partial(
        pl.kernel,
        out_type=jax.ShapeDtypeStruct((NC, N_NODES, DH), jnp.float32),
        mesh=mesh,
        scratch_types=[
            pltpu.VMEM_SHARED((ACC_ROWS, DH), jnp.float32),
            pltpu.VMEM((2, N_CH, CH), jnp.int32),
            pltpu.VMEM((2, N_CH, CH), jnp.int32),
            pltpu.VMEM((2, N_CH * CH, DH), jnp.float32),
            pltpu.SemaphoreType.DMA((2, N_CH)),
            pltpu.SemaphoreType.DMA((2, N_CH)),
            pltpu.SemaphoreType.DMA((2,)),
        ],
        compiler_params=pltpu.CompilerParams(use_tc_tiling_on_sc=False),
    )
    def agg(table, edges, out, acc, src_a, dst_a, rows_a, gsem, ssem, isem):
        c = lax.axis_index("c")
        s = lax.axis_index("s")
        tbl = table.at[c]

        zero = jnp.zeros((16,), jnp.float32)
        for r in range(ZCH):
            rows_a[0, r, pl.ds(0, 16)] = zero
        z0 = s * (ACC_ROWS // NS)

        def zloop(j, _):
            pltpu.sync_copy(
                rows_a.at[0, pl.ds(0, ZCH)], acc.at[pl.ds(z0 + j * ZCH, ZCH)]
            )
            return 0

        lax.fori_loop(0, (ACC_ROWS // NS) // ZCH, zloop, 0)
        plsc.subcore_barrier()

        nblocks = (NBLK - s + NS - 1) // NS

        pltpu.sync_copy(edges.at[0, pl.ds(s * N_CH, N_CH)], src_a.at[0])
        pltpu.sync_copy(edges.at[1, pl.ds(s * N_CH, N_CH)], dst_a.at[0])

        def outer(i, _):
            p = i & 1
            q = 1 - p
            gd = []
            for b in range(N_CH):
                gd.append(pltpu.async_copy(
                    tbl.at[src_a.at[p, b]],
                    rows_a.at[p, pl.ds(b * CH, CH)],
                    gsem.at[p, b],
                ))

            @pl.when(i >= 1)
            def _():
                for b in range(N_CH):
                    pltpu.make_async_copy(
                        rows_a.at[q, pl.ds(b * CH, CH)],
                        acc.at[dst_a.at[q, b]],
                        ssem.at[q, b],
                    ).wait()

            @pl.when(i + 1 < nblocks)
            def _():
                r0n = (s + (i + 1) * NS) * N_CH
                pltpu.async_copy(
                    edges.at[0, pl.ds(r0n, N_CH)], src_a.at[q], isem.at[q]
                )
                pltpu.async_copy(
                    edges.at[1, pl.ds(r0n, N_CH)], dst_a.at[q], isem.at[q]
                )

            for b in range(N_CH):
                gd[b].wait()
                pltpu.async_copy(
                    rows_a.at[p, pl.ds(b * CH, CH)],
                    acc.at[dst_a.at[p, b]],
                    ssem.at[p, b],
                    add=True,
                )

            @pl.when(i + 1 < nblocks)
            def _():
                r0n = (s + (i + 1) * NS) * N_CH
                pltpu.make_async_copy(
                    edges.at[0, pl.ds(r0n, N_CH)], src_a.at[q], isem.at[q]
                ).wait()
                pltpu.make_async_copy(
                    edges.at[1, pl.ds(r0n, N_CH)], dst_a.at[q], isem.at[q]
                ).wait()

            return 0

        lax.fori_loop(0, nblocks, outer, 0)
        last = (nblocks - 1) & 1
        for b in range(N_CH):
            pltpu.make_async_copy(
                rows_a.at[last, pl.ds(b * CH, CH)],
                acc.at[dst_a.at[last, b]],
                ssem.at[last, b],
            ).wait()
        plsc.subcore_barrier()

        n_chunks = (N_OUT_CHUNKS - s + NS - 1) // NS

        def cloop(i, _):
            r0 = (s + i * NS) * OUT_CH
            pltpu.sync_copy(
                acc.at[pl.ds(r0, OUT_CH)], out.at[c, pl.ds(r0, OUT_CH)]
            )
            return 0

        lax.fori_loop(0, n_chunks, cloop, 0)

    return agg(table3, edge3)


_BN = 2000
_NBLK = N_NODES // _BN


def _mlp_body(h_ref, w1_ref, b1_ref, w2_ref, o_ref):
    h = jnp.concatenate([h_ref[0], h_ref[1]], axis=1)
    t = jnp.dot(h, w1_ref[...], preferred_element_type=jnp.float32)
    t = jnp.maximum(t + b1_ref[...], 0.0)
    y = jnp.dot(t, w2_ref[...], preferred_element_type=jnp.float32)
    o_ref[0] = y[:, :DH]
    o_ref[1] = y[:, DH:]


def _mlp(h1s, W1p, b1r, W2p):
    return pl.pallas_call(
        _mlp_body,
        grid=(_NBLK,),
        in_specs=[
            pl.BlockSpec((NC, _BN, DH), lambda i: (0, i, 0)),
            pl.BlockSpec((D_PAD, 48), lambda i: (0, 0)),
            pl.BlockSpec((1, 48), lambda i: (0, 0)),
            pl.BlockSpec((48, D_PAD), lambda i: (0, 0)),
        ],
        out_specs=pl.BlockSpec((NC, _BN, DH), lambda i: (0, i, 0)),
        out_shape=jax.ShapeDtypeStruct((NC, N_NODES, DH), jnp.float32),
    )(h1s, W1p, b1r, W2p)


_CNT = float(N_NODES * 24)


def _finale_body(h_ref, b2_ref, o_ref, st_ref):
    j = pl.program_id(0)
    i = pl.program_id(1)
    x = jnp.concatenate([h_ref[0], h_ref[1]], axis=1) + b2_ref[...]

    @pl.when(j == 0)
    def _():
        @pl.when(i == 0)
        def _():
            st_ref[0] = 0.0
            st_ref[1] = 0.0

        col = lax.broadcasted_iota(jnp.int32, (_BN, D_PAD), 1)
        xs = jnp.where(col < 24, x, 0.0)
        st_ref[0] += jnp.sum(xs)
        st_ref[1] += jnp.sum(xs * xs)

    @pl.when(j == 1)
    def _():
        mu = st_ref[0] / _CNT
        var = (st_ref[1] - _CNT * mu * mu) / (_CNT - 1.0)
        inv = lax.rsqrt(var)
        o_ref[...] = (x[:, :24] - mu) * inv


def _finale(h2s, b2r):
    return pl.pallas_call(
        _finale_body,
        grid=(2, _NBLK),
        in_specs=[
            pl.BlockSpec((NC, _BN, DH), lambda j, i: (0, i, 0)),
            pl.BlockSpec((1, D_PAD), lambda j, i: (0, 0)),
        ],
        out_specs=pl.BlockSpec((_BN, 24), lambda j, i: (i, 0)),
        out_shape=jax.ShapeDtypeStruct((N_NODES, 24), jnp.float32),
        scratch_shapes=[pltpu.SMEM((2,), jnp.float32)],
    )(h2s, b2r)


def kernel(features, edge_index, W1, b1, W2, b2):
    n, d = features.shape
    assert n == N_NODES and d == 24 and edge_index.shape == (2, N_EDGES)

    fpad = jnp.pad(features, ((0, 0), (0, D_PAD - 24)))
    ftab = jnp.stack([fpad[:, :DH], fpad[:, DH:]], axis=0)
    W1p = jnp.pad(W1, ((0, D_PAD - 24), (0, 0)))
    W2p = jnp.pad(W2, ((0, 0), (0, D_PAD - 24)))
    b1r = b1.reshape(1, 48)
    b2r = jnp.pad(b2, (0, D_PAD - 24)).reshape(1, D_PAD)

    edge3 = edge_index.reshape(NC, N_IDX_ROWS, CH)

    h1s = _sc_aggregate(ftab, edge3)
    ys = _mlp(h1s, W1p, b1r, W2p)
    h2s = _sc_aggregate(ys, edge3)
    return _finale(h2s, b2r)


# speedup vs baseline: 1.1620x; 1.0667x over previous
import functools

import jax
import jax.numpy as jnp
from jax import lax
from jax.experimental import pallas as pl
from jax.experimental.pallas import tpu as pltpu
from jax.experimental.pallas import tpu_sc as plsc

N_NODES = 100000
N_EDGES = 3200000
D_PAD = 32
DH = 16

NC = 2
NS = 16

ACC_ROWS = 100352
CH = 160
N_CH = 5
BLK = CH * N_CH
NBLK = N_EDGES // BLK
N_IDX_ROWS = N_EDGES // CH
ZCH = 128
OUT_CH = 1000
N_OUT_CHUNKS = N_NODES // OUT_CH


def _sc_aggregate(table3, edge3):
    mesh = plsc.VectorSubcoreMesh(core_axis_name="c", subcore_axis_name="s")

    @functools.partial(
        pl.kernel,
        out_type=jax.ShapeDtypeStruct((NC, N_NODES, DH), jnp.float32),
        mesh=mesh,
        scratch_types=[
            pltpu.VMEM_SHARED((ACC_ROWS, DH), jnp.float32),
            pltpu.VMEM((2, N_CH, CH), jnp.int32),
            pltpu.VMEM((2, N_CH, CH), jnp.int32),
            pltpu.VMEM((2, N_CH * CH, DH), jnp.float32),
            pltpu.SemaphoreType.DMA((2, N_CH)),
            pltpu.SemaphoreType.DMA((2, N_CH)),
            pltpu.SemaphoreType.DMA((2,)),
        ],
        compiler_params=pltpu.CompilerParams(use_tc_tiling_on_sc=False),
    )
    def agg(table, edges, out, acc, src_a, dst_a, rows_a, gsem, ssem, isem):
        c = lax.axis_index("c")
        s = lax.axis_index("s")
        tbl = table.at[c]

        zero = jnp.zeros((16,), jnp.float32)
        for r in range(ZCH):
            rows_a[0, r, pl.ds(0, 16)] = zero
        z0 = s * (ACC_ROWS // NS)

        def zloop(j, _):
            pltpu.sync_copy(
                rows_a.at[0, pl.ds(0, ZCH)], acc.at[pl.ds(z0 + j * ZCH, ZCH)]
            )
            return 0

        lax.fori_loop(0, (ACC_ROWS // NS) // ZCH, zloop, 0)
        plsc.subcore_barrier()

        nblocks = (NBLK - s + NS - 1) // NS

        pltpu.sync_copy(edges.at[0, pl.ds(s * N_CH, N_CH)], src_a.at[0])
        pltpu.sync_copy(edges.at[1, pl.ds(s * N_CH, N_CH)], dst_a.at[0])

        def outer(i, _):
            p = i & 1
            q = 1 - p
            gd = []
            for b in range(N_CH):
                gd.append(pltpu.async_copy(
                    tbl.at[src_a.at[p, b]],
                    rows_a.at[p, pl.ds(b * CH, CH)],
                    gsem.at[p, b],
                ))

            @pl.when(i >= 1)
            def _():
                for b in range(N_CH):
                    pltpu.make_async_copy(
                        rows_a.at[q, pl.ds(b * CH, CH)],
                        acc.at[dst_a.at[q, b]],
                        ssem.at[q, b],
                    ).wait()

            @pl.when(i + 1 < nblocks)
            def _():
                r0n = (s + (i + 1) * NS) * N_CH
                pltpu.async_copy(
                    edges.at[0, pl.ds(r0n, N_CH)], src_a.at[q], isem.at[q]
                )
                pltpu.async_copy(
                    edges.at[1, pl.ds(r0n, N_CH)], dst_a.at[q], isem.at[q]
                )

            for b in range(N_CH):
                gd[b].wait()
                pltpu.async_copy(
                    rows_a.at[p, pl.ds(b * CH, CH)],
                    acc.at[dst_a.at[p, b]],
                    ssem.at[p, b],
                    add=True,
                )

            @pl.when(i + 1 < nblocks)
            def _():
                r0n = (s + (i + 1) * NS) * N_CH
                pltpu.make_async_copy(
                    edges.at[0, pl.ds(r0n, N_CH)], src_a.at[q], isem.at[q]
                ).wait()
                pltpu.make_async_copy(
                    edges.at[1, pl.ds(r0n, N_CH)], dst_a.at[q], isem.at[q]
                ).wait()

            return 0

        lax.fori_loop(0, nblocks, outer, 0)
        last = (nblocks - 1) & 1
        for b in range(N_CH):
            pltpu.make_async_copy(
                rows_a.at[last, pl.ds(b * CH, CH)],
                acc.at[dst_a.at[last, b]],
                ssem.at[last, b],
            ).wait()
        plsc.subcore_barrier()

        n_chunks = (N_OUT_CHUNKS - s + NS - 1) // NS

        def cloop(i, _):
            r0 = (s + i * NS) * OUT_CH
            pltpu.sync_copy(
                acc.at[pl.ds(r0, OUT_CH)], out.at[c, pl.ds(r0, OUT_CH)]
            )
            return 0

        lax.fori_loop(0, n_chunks, cloop, 0)

    return agg(table3, edge3)


_BN = 2000
_NBLK = N_NODES // _BN


def _mlp_body(h_ref, w1_ref, b1_ref, w2_ref, o_ref):
    h = jnp.concatenate([h_ref[0], h_ref[1]], axis=1)
    t = jnp.dot(h, w1_ref[...], preferred_element_type=jnp.float32)
    t = jnp.maximum(t + b1_ref[...], 0.0)
    y = jnp.dot(t, w2_ref[...], preferred_element_type=jnp.float32)
    o_ref[0] = y[:, :DH]
    o_ref[1] = y[:, DH:]


def _mlp(h1s, W1p, b1r, W2p):
    return pl.pallas_call(
        _mlp_body,
        grid=(_NBLK,),
        in_specs=[
            pl.BlockSpec((NC, _BN, DH), lambda i: (0, i, 0)),
            pl.BlockSpec((D_PAD, 48), lambda i: (0, 0)),
            pl.BlockSpec((1, 48), lambda i: (0, 0)),
            pl.BlockSpec((48, D_PAD), lambda i: (0, 0)),
        ],
        out_specs=pl.BlockSpec((NC, _BN, DH), lambda i: (0, i, 0)),
        out_shape=jax.ShapeDtypeStruct((NC, N_NODES, DH), jnp.float32),
    )(h1s, W1p, b1r, W2p)


def _stats_body(h_ref, b2_ref, sum_ref, sq_ref):
    i = pl.program_id(0)
    x = jnp.concatenate([h_ref[0], h_ref[1]], axis=1) + b2_ref[...]
    col = lax.broadcasted_iota(jnp.int32, (_BN, D_PAD), 1)
    xs = jnp.where(col < 24, x, 0.0)

    @pl.when(i == 0)
    def _():
        sum_ref[...] = jnp.zeros((1, 1), jnp.float32)
        sq_ref[...] = jnp.zeros((1, 1), jnp.float32)

    sum_ref[...] += jnp.full((1, 1), jnp.sum(xs), jnp.float32)
    sq_ref[...] += jnp.full((1, 1), jnp.sum(xs * xs), jnp.float32)


def _stats(h2s, b2r):
    return pl.pallas_call(
        _stats_body,
        grid=(_NBLK,),
        in_specs=[
            pl.BlockSpec((NC, _BN, DH), lambda i: (0, i, 0)),
            pl.BlockSpec((1, D_PAD), lambda i: (0, 0)),
        ],
        out_specs=[
            pl.BlockSpec((1, 1), lambda i: (0, 0)),
            pl.BlockSpec((1, 1), lambda i: (0, 0)),
        ],
        out_shape=[
            jax.ShapeDtypeStruct((1, 1), jnp.float32),
            jax.ShapeDtypeStruct((1, 1), jnp.float32),
        ],
    )(h2s, b2r)


def _apply_body(h_ref, b2_ref, mu_ref, inv_ref, o_ref):
    x = jnp.concatenate([h_ref[0], h_ref[1][:, :8]], axis=1)
    xb = x + b2_ref[...][:, :24]
    o_ref[...] = (xb - mu_ref[0, 0]) * inv_ref[0, 0]


def _apply(h2s, b2r, mu, inv):
    return pl.pallas_call(
        _apply_body,
        grid=(_NBLK,),
        in_specs=[
            pl.BlockSpec((NC, _BN, DH), lambda i: (0, i, 0)),
            pl.BlockSpec((1, D_PAD), lambda i: (0, 0)),
            pl.BlockSpec((1, 1), lambda i: (0, 0)),
            pl.BlockSpec((1, 1), lambda i: (0, 0)),
        ],
        out_specs=pl.BlockSpec((_BN, 24), lambda i: (i, 0)),
        out_shape=jax.ShapeDtypeStruct((N_NODES, 24), jnp.float32),
    )(h2s, b2r, mu, inv)


def kernel(features, edge_index, W1, b1, W2, b2):
    n, d = features.shape
    assert n == N_NODES and d == 24 and edge_index.shape == (2, N_EDGES)

    fpad = jnp.pad(features, ((0, 0), (0, D_PAD - 24)))
    ftab = jnp.stack([fpad[:, :DH], fpad[:, DH:]], axis=0)
    W1p = jnp.pad(W1, ((0, D_PAD - 24), (0, 0)))
    W2p = jnp.pad(W2, ((0, 0), (0, D_PAD - 24)))
    b1r = b1.reshape(1, 48)
    b2r = jnp.pad(b2, (0, D_PAD - 24)).reshape(1, D_PAD)

    edge3 = edge_index.reshape(NC, N_IDX_ROWS, CH)

    h1s = _sc_aggregate(ftab, edge3)
    ys = _mlp(h1s, W1p, b1r, W2p)
    h2s = _sc_aggregate(ys, edge3)

    ssum, ssq = _stats(h2s, b2r)
    cnt = jnp.float32(N_NODES * 24)
    mu = ssum / cnt
    var = (ssq - cnt * mu * mu) / (cnt - 1.0)
    inv = lax.rsqrt(var)
    return _apply(h2s, b2r, mu, inv)
